# baseline (speedup 1.0000x reference)
import jax
import jax.numpy as jnp
from jax import lax
from jax.experimental import pallas as pl
from jax.experimental.pallas import tpu as pltpu
from jax.experimental.pallas import tpu_sc as plsc

N = 10000
E = 320000
NC = 2
NS = 16
CH = 128
NW = NC * NS
R = E // CH
ROWS_MAIN = 80
ROWS_LAST = R - ROWS_MAIN * (NW - 1)
DEPTH = 3
PAD_ROWS = 240
NACC = N + PAD_ROWS
BLK = 2000


def _seg_sum_call(table, src2, dst2, zeros_d, zeros_c=None, ones_c=None):
    D = table.shape[1]
    with_cnt = zeros_c is not None
    K = 2
    mesh = plsc.VectorSubcoreMesh(
        core_axis_name="c", subcore_axis_name="s",
        num_cores=NC, num_subcores=NS)

    def pipeline(table_h, src_h, dst_h, out_h, cnt_h, sidx, didx, rows,
                 gsem, asem, acc, ones_v, cacc, zd_h, zc_h, on_h):
        c = lax.axis_index("c")
        s = lax.axis_index("s")
        w = c * NS + s
        zrows = NACC // NS
        pltpu.sync_copy(zd_h.at[pl.ds(s * zrows, zrows)],
                        acc.at[pl.ds(s * zrows, zrows)])
        if with_cnt:
            pltpu.sync_copy(zc_h.at[pl.ds(s * zrows, zrows)],
                            cacc.at[pl.ds(s * zrows, zrows)])
            pltpu.sync_copy(on_h, ones_v)
        plsc.subcore_barrier()

        base = w * ROWS_MAIN
        last = w == NW - 1
        nrows = jnp.where(last, ROWS_LAST, ROWS_MAIN)
        nblocks = nrows // K

        @pl.when(jnp.logical_not(last))
        def _():
            pltpu.sync_copy(src_h.at[pl.ds(base, ROWS_MAIN)], sidx)
            pltpu.sync_copy(dst_h.at[pl.ds(base, ROWS_MAIN)], didx)

        @pl.when(last)
        def _():
            pltpu.sync_copy(src_h.at[pl.ds(base, ROWS_LAST)],
                            sidx.at[pl.ds(0, ROWS_LAST)])
            pltpu.sync_copy(dst_h.at[pl.ds(base, ROWS_LAST)],
                            didx.at[pl.ds(0, ROWS_LAST)])

        def fire(b, p):
            for j in range(K):
                pltpu.async_copy(table_h.at[sidx.at[b * K + j]],
                                 rows.at[p, j], gsem)

        def wait_gathers(b, p):
            for j in range(K):
                pltpu.make_async_copy(table_h.at[sidx.at[b * K + j]],
                                      rows.at[p, j], gsem).wait()

        def fire_adds(b, p):
            for j in range(K):
                pltpu.async_copy(rows.at[p, j], acc.at[didx.at[b * K + j]],
                                 asem, add=True)
            if with_cnt:
                for j in range(K):
                    pltpu.async_copy(ones_v, cacc.at[didx.at[b * K + j]],
                                     asem, add=True)

        def drain_adds(b, p):
            for j in range(K):
                pltpu.make_async_copy(rows.at[p, j],
                                      acc.at[didx.at[b * K + j]],
                                      asem).wait()
            if with_cnt:
                for j in range(K):
                    pltpu.make_async_copy(ones_v,
                                          cacc.at[didx.at[b * K + j]],
                                          asem).wait()

        fire(0, 0)
        fire(1, 1)

        nphases = nblocks + 1

        @pl.loop(0, (nphases + DEPTH - 1) // DEPTH)
        def _(i):
            for k in range(DEPTH):
                b = i * DEPTH + k

                @pl.when(b < nblocks)
                def _(b=b, k=k):
                    wait_gathers(b, k)
                    fire_adds(b, k)

                @pl.when(jnp.logical_and(b >= 1, b <= nblocks))
                def _(b=b, k=k):
                    drain_adds(b - 1, (k - 1) % DEPTH)

                @pl.when(b + DEPTH - 1 < nblocks)
                def _(b=b, k=k):
                    fire(b + DEPTH - 1, (k + DEPTH - 1) % DEPTH)

        plsc.subcore_barrier()
        orows = NACC // NS
        tail = N - (NS - 1) * orows

        @pl.when(s < NS - 1)
        def _():
            pltpu.sync_copy(acc.at[pl.ds(s * orows, orows)],
                            out_h.at[pl.ds(c * N + s * orows, orows)])
            if with_cnt:
                pltpu.sync_copy(cacc.at[pl.ds(s * orows, orows)],
                                cnt_h.at[pl.ds(c * N + s * orows, orows)])

        @pl.when(s == NS - 1)
        def _():
            pltpu.sync_copy(acc.at[pl.ds((NS - 1) * orows, tail)],
                            out_h.at[pl.ds(c * N + (NS - 1) * orows, tail)])
            if with_cnt:
                pltpu.sync_copy(cacc.at[pl.ds((NS - 1) * orows, tail)],
                                cnt_h.at[pl.ds(c * N + (NS - 1) * orows, tail)])

    if with_cnt:
        def body(table_h, src_h, dst_h, zd_h, zc_h, on_h, out_h, cnt_h,
                 sidx, didx, rows, gsem, asem, acc, ones_v, cacc):
            pipeline(table_h, src_h, dst_h, out_h, cnt_h, sidx, didx, rows,
                     gsem, asem, acc, ones_v, cacc, zd_h, zc_h, on_h)

        out_type = (jax.ShapeDtypeStruct((NC * N, D), jnp.float32),
                    jax.ShapeDtypeStruct((NC * N, 16), jnp.float32))
        scratch = [
            pltpu.VMEM((ROWS_MAIN, CH), jnp.int32),
            pltpu.VMEM((ROWS_MAIN, CH), jnp.int32),
            pltpu.VMEM((DEPTH, K, CH, D), jnp.float32),
            pltpu.SemaphoreType.DMA,
            pltpu.SemaphoreType.DMA,
            pltpu.VMEM_SHARED((NACC, D), jnp.float32),
            pltpu.VMEM((CH, 16), jnp.float32),
            pltpu.VMEM_SHARED((NACC, 16), jnp.float32),
        ]
        operands = (table, src2, dst2, zeros_d, zeros_c, ones_c)
    else:
        def body(table_h, src_h, dst_h, zd_h, out_h,
                 sidx, didx, rows, gsem, asem, acc):
            pipeline(table_h, src_h, dst_h, out_h, None, sidx, didx, rows,
                     gsem, asem, acc, None, None, zd_h, None, None)

        out_type = jax.ShapeDtypeStruct((NC * N, D), jnp.float32)
        scratch = [
            pltpu.VMEM((ROWS_MAIN, CH), jnp.int32),
            pltpu.VMEM((ROWS_MAIN, CH), jnp.int32),
            pltpu.VMEM((DEPTH, K, CH, D), jnp.float32),
            pltpu.SemaphoreType.DMA,
            pltpu.SemaphoreType.DMA,
            pltpu.VMEM_SHARED((NACC, D), jnp.float32),
        ]
        operands = (table, src2, dst2, zeros_d)

    kern = pl.kernel(
        body,
        out_type=out_type,
        mesh=mesh,
        compiler_params=pltpu.CompilerParams(use_tc_tiling_on_sc=False),
        scratch_types=scratch,
    )
    return kern(*operands)


def _tc_first(x, Wl, Wr, b):
    Do = Wl.shape[1]

    def body(x_ref, wl_ref, wr_ref, b_ref, y_ref, r_ref):
        xv = x_ref[...]
        y_ref[...] = jnp.dot(xv, wl_ref[...], preferred_element_type=jnp.float32)
        r_ref[...] = (jnp.dot(xv, wr_ref[...], preferred_element_type=jnp.float32)
                      + b_ref[...][None, :])

    return pl.pallas_call(
        body,
        grid=(N // BLK,),
        in_specs=[
            pl.BlockSpec((BLK, 128), lambda i: (i, 0)),
            pl.BlockSpec(Wl.shape, lambda i: (0, 0)),
            pl.BlockSpec(Wr.shape, lambda i: (0, 0)),
            pl.BlockSpec(b.shape, lambda i: (0,)),
        ],
        out_specs=(pl.BlockSpec((BLK, Do), lambda i: (i, 0)),
                   pl.BlockSpec((BLK, Do), lambda i: (i, 0))),
        out_shape=(jax.ShapeDtypeStruct((N, Do), jnp.float32),
                   jax.ShapeDtypeStruct((N, Do), jnp.float32)),
    )(x, Wl, Wr, b)


def _tc_mid1(p, cntp, r, Wl, Wr, b):
    Di = p.shape[1]
    Do = Wl.shape[1]

    def body(p0_ref, p1_ref, c0_ref, c1_ref, r_ref, wl_ref, wr_ref, b_ref,
             y_ref, rn_ref, inv_ref):
        sacc = p0_ref[...] + p1_ref[...]
        cnt = c0_ref[...][:, 0:1] + c1_ref[...][:, 0:1]
        inv = 1.0 / jnp.maximum(cnt, 1.0)
        inv_ref[...] = inv
        h = jnp.maximum(sacc * inv + r_ref[...], 0.0)
        y_ref[...] = jnp.dot(h, wl_ref[...], preferred_element_type=jnp.float32)
        rn_ref[...] = (jnp.dot(h, wr_ref[...], preferred_element_type=jnp.float32)
                       + b_ref[...][None, :])

    nb = N // BLK
    return pl.pallas_call(
        body,
        grid=(nb,),
        in_specs=[
            pl.BlockSpec((BLK, Di), lambda i: (i, 0)),
            pl.BlockSpec((BLK, Di), lambda i: (i + nb, 0)),
            pl.BlockSpec((BLK, 16), lambda i: (i, 0)),
            pl.BlockSpec((BLK, 16), lambda i: (i + nb, 0)),
            pl.BlockSpec((BLK, Di), lambda i: (i, 0)),
            pl.BlockSpec(Wl.shape, lambda i: (0, 0)),
            pl.BlockSpec(Wr.shape, lambda i: (0, 0)),
            pl.BlockSpec(b.shape, lambda i: (0,)),
        ],
        out_specs=(pl.BlockSpec((BLK, Do), lambda i: (i, 0)),
                   pl.BlockSpec((BLK, Do), lambda i: (i, 0)),
                   pl.BlockSpec((BLK, 1), lambda i: (i, 0))),
        out_shape=(jax.ShapeDtypeStruct((N, Do), jnp.float32),
                   jax.ShapeDtypeStruct((N, Do), jnp.float32),
                   jax.ShapeDtypeStruct((N, 1), jnp.float32)),
    )(p, p, cntp, cntp, r, Wl, Wr, b)


def _tc_mid2(p, inv, r, Wl, Wr, b):
    Di = p.shape[1]
    Do = Wl.shape[1]

    def body(p0_ref, p1_ref, i_ref, r_ref, wl_ref, wr_ref, b_ref,
             y_ref, rn_ref):
        sacc = p0_ref[...] + p1_ref[...]
        h = jnp.maximum(sacc * i_ref[...] + r_ref[...], 0.0)
        y_ref[...] = jnp.dot(h, wl_ref[...], preferred_element_type=jnp.float32)
        rn_ref[...] = (jnp.dot(h, wr_ref[...], preferred_element_type=jnp.float32)
                       + b_ref[...][None, :])

    nb = N // BLK
    return pl.pallas_call(
        body,
        grid=(nb,),
        in_specs=[
            pl.BlockSpec((BLK, Di), lambda i: (i, 0)),
            pl.BlockSpec((BLK, Di), lambda i: (i + nb, 0)),
            pl.BlockSpec((BLK, 1), lambda i: (i, 0)),
            pl.BlockSpec((BLK, Di), lambda i: (i, 0)),
            pl.BlockSpec(Wl.shape, lambda i: (0, 0)),
            pl.BlockSpec(Wr.shape, lambda i: (0, 0)),
            pl.BlockSpec(b.shape, lambda i: (0,)),
        ],
        out_specs=(pl.BlockSpec((BLK, Do), lambda i: (i, 0)),
                   pl.BlockSpec((BLK, Do), lambda i: (i, 0))),
        out_shape=(jax.ShapeDtypeStruct((N, Do), jnp.float32),
                   jax.ShapeDtypeStruct((N, Do), jnp.float32)),
    )(p, p, inv, r, Wl, Wr, b)


def _tc_final(p, inv, r, Wreg, breg):
    Di = p.shape[1]

    def body(p0_ref, p1_ref, i_ref, r_ref, w_ref, b_ref, o_ref):
        sacc = p0_ref[...] + p1_ref[...]
        h = jnp.maximum(sacc * i_ref[...] + r_ref[...], 0.0)
        o_ref[...] = (jnp.dot(h, w_ref[...], preferred_element_type=jnp.float32)
                      + b_ref[...][None, :])

    nb = N // BLK
    return pl.pallas_call(
        body,
        grid=(nb,),
        in_specs=[
            pl.BlockSpec((BLK, Di), lambda i: (i, 0)),
            pl.BlockSpec((BLK, Di), lambda i: (i + nb, 0)),
            pl.BlockSpec((BLK, 1), lambda i: (i, 0)),
            pl.BlockSpec((BLK, Di), lambda i: (i, 0)),
            pl.BlockSpec(Wreg.shape, lambda i: (0, 0)),
            pl.BlockSpec(breg.shape, lambda i: (0,)),
        ],
        out_specs=pl.BlockSpec((BLK, 1), lambda i: (i, 0)),
        out_shape=jax.ShapeDtypeStruct((N, 1), jnp.float32),
    )(p, p, inv, r, Wreg, breg)


def kernel(x, edge_index, W1l, b1l, W1r, W2l, b2l, W2r, W3l, b3l, W3r,
           Wreg, breg):
    src2 = edge_index[0].reshape(R, CH)
    dst2 = edge_index[1].reshape(R, CH)

    zeros64 = jnp.zeros((NACC, 64), jnp.float32)
    zeros32 = jnp.zeros((NACC, 32), jnp.float32)
    zeros16 = jnp.zeros((NACC, 16), jnp.float32)
    ones16 = jnp.ones((CH, 16), jnp.float32)

    y1, r1 = _tc_first(x, W1l, W1r, b1l)
    p1, cntp = _seg_sum_call(y1, src2, dst2, zeros64, zeros16, ones16)
    y2, r2, inv = _tc_mid1(p1, cntp, r1, W2l, W2r, b2l)
    p2 = _seg_sum_call(y2, src2, dst2, zeros64)
    y3, r3 = _tc_mid2(p2, inv, r2, W3l, W3r, b3l)
    p3 = _seg_sum_call(y3, src2, dst2, zeros32)
    return jnp.squeeze(_tc_final(p3, inv, r3, Wreg, breg), axis=-1)


# speedup vs baseline: 1.0344x; 1.0344x over previous
import jax
import jax.numpy as jnp
from jax import lax
from jax.experimental import pallas as pl
from jax.experimental.pallas import tpu as pltpu
from jax.experimental.pallas import tpu_sc as plsc

N = 10000
E = 320000
NC = 2
NS = 16
CH = 128
NW = NC * NS
R = E // CH
ROWS_MAIN = 80
ROWS_LAST = R - ROWS_MAIN * (NW - 1)
DEPTH = 3
PAD_ROWS = 240
NACC = N + PAD_ROWS


def _seg_sum_call(table, eidx2, zeros_d, zeros_c=None, ones_c=None):
    D = table.shape[1]
    with_cnt = zeros_c is not None
    K = 2
    mesh = plsc.VectorSubcoreMesh(
        core_axis_name="c", subcore_axis_name="s",
        num_cores=NC, num_subcores=NS)

    def pipeline(table_h, eidx_h, out_h, cnt_h, sidx, didx, rows,
                 gsem, asem, acc, ones_v, cacc, zd_h, zc_h, on_h):
        c = lax.axis_index("c")
        s = lax.axis_index("s")
        w = c * NS + s
        zrows = NACC // NS
        pltpu.sync_copy(zd_h.at[pl.ds(s * zrows, zrows)],
                        acc.at[pl.ds(s * zrows, zrows)])
        if with_cnt:
            pltpu.sync_copy(zc_h.at[pl.ds(s * zrows, zrows)],
                            cacc.at[pl.ds(s * zrows, zrows)])
            pltpu.sync_copy(on_h, ones_v)
        plsc.subcore_barrier()

        base = w * ROWS_MAIN
        last = w == NW - 1
        nrows = jnp.where(last, ROWS_LAST, ROWS_MAIN)
        nblocks = nrows // K

        @pl.when(jnp.logical_not(last))
        def _():
            pltpu.sync_copy(eidx_h.at[0, pl.ds(base, ROWS_MAIN)], sidx)
            pltpu.sync_copy(eidx_h.at[1, pl.ds(base, ROWS_MAIN)], didx)

        @pl.when(last)
        def _():
            pltpu.sync_copy(eidx_h.at[0, pl.ds(base, ROWS_LAST)],
                            sidx.at[pl.ds(0, ROWS_LAST)])
            pltpu.sync_copy(eidx_h.at[1, pl.ds(base, ROWS_LAST)],
                            didx.at[pl.ds(0, ROWS_LAST)])

        def fire(b, p):
            for j in range(K):
                pltpu.async_copy(table_h.at[sidx.at[b * K + j]],
                                 rows.at[p, j], gsem)

        def wait_gathers(b, p):
            for j in range(K):
                pltpu.make_async_copy(table_h.at[sidx.at[b * K + j]],
                                      rows.at[p, j], gsem).wait()

        def fire_adds(b, p):
            for j in range(K):
                pltpu.async_copy(rows.at[p, j], acc.at[didx.at[b * K + j]],
                                 asem, add=True)
            if with_cnt:
                for j in range(K):
                    pltpu.async_copy(ones_v, cacc.at[didx.at[b * K + j]],
                                     asem, add=True)

        def drain_adds(b, p):
            for j in range(K):
                pltpu.make_async_copy(rows.at[p, j],
                                      acc.at[didx.at[b * K + j]],
                                      asem).wait()
            if with_cnt:
                for j in range(K):
                    pltpu.make_async_copy(ones_v,
                                          cacc.at[didx.at[b * K + j]],
                                          asem).wait()

        fire(0, 0)
        fire(1, 1)

        nphases = nblocks + 1

        @pl.loop(0, (nphases + DEPTH - 1) // DEPTH)
        def _(i):
            for k in range(DEPTH):
                b = i * DEPTH + k

                @pl.when(b < nblocks)
                def _(b=b, k=k):
                    wait_gathers(b, k)
                    fire_adds(b, k)

                @pl.when(jnp.logical_and(b >= 1, b <= nblocks))
                def _(b=b, k=k):
                    drain_adds(b - 1, (k - 1) % DEPTH)

                @pl.when(b + DEPTH - 1 < nblocks)
                def _(b=b, k=k):
                    fire(b + DEPTH - 1, (k + DEPTH - 1) % DEPTH)

        plsc.subcore_barrier()
        orows = NACC // NS
        pltpu.sync_copy(acc.at[pl.ds(s * orows, orows)],
                        out_h.at[pl.ds(c * NACC + s * orows, orows)])
        if with_cnt:
            pltpu.sync_copy(cacc.at[pl.ds(s * orows, orows)],
                            cnt_h.at[pl.ds(c * NACC + s * orows, orows)])

    if with_cnt:
        def body(table_h, eidx_h, zd_h, zc_h, on_h, out_h, cnt_h,
                 sidx, didx, rows, gsem, asem, acc, ones_v, cacc):
            pipeline(table_h, eidx_h, out_h, cnt_h, sidx, didx, rows,
                     gsem, asem, acc, ones_v, cacc, zd_h, zc_h, on_h)

        out_type = (jax.ShapeDtypeStruct((NC * NACC, D), jnp.float32),
                    jax.ShapeDtypeStruct((NC * NACC, 16), jnp.float32))
        scratch = [
            pltpu.VMEM((ROWS_MAIN, CH), jnp.int32),
            pltpu.VMEM((ROWS_MAIN, CH), jnp.int32),
            pltpu.VMEM((DEPTH, K, CH, D), jnp.float32),
            pltpu.SemaphoreType.DMA,
            pltpu.SemaphoreType.DMA,
            pltpu.VMEM_SHARED((NACC, D), jnp.float32),
            pltpu.VMEM((CH, 16), jnp.float32),
            pltpu.VMEM_SHARED((NACC, 16), jnp.float32),
        ]
        operands = (table, eidx2, zeros_d, zeros_c, ones_c)
    else:
        def body(table_h, eidx_h, zd_h, out_h,
                 sidx, didx, rows, gsem, asem, acc):
            pipeline(table_h, eidx_h, out_h, None, sidx, didx, rows,
                     gsem, asem, acc, None, None, zd_h, None, None)

        out_type = jax.ShapeDtypeStruct((NC * NACC, D), jnp.float32)
        scratch = [
            pltpu.VMEM((ROWS_MAIN, CH), jnp.int32),
            pltpu.VMEM((ROWS_MAIN, CH), jnp.int32),
            pltpu.VMEM((DEPTH, K, CH, D), jnp.float32),
            pltpu.SemaphoreType.DMA,
            pltpu.SemaphoreType.DMA,
            pltpu.VMEM_SHARED((NACC, D), jnp.float32),
        ]
        operands = (table, eidx2, zeros_d)

    kern = pl.kernel(
        body,
        out_type=out_type,
        mesh=mesh,
        compiler_params=pltpu.CompilerParams(use_tc_tiling_on_sc=False),
        scratch_types=scratch,
    )
    return kern(*operands)


def _tc_first(x, Wl, Wr, b):
    Do = Wl.shape[1]

    def body(x_ref, wl_ref, wr_ref, b_ref, y_ref, r_ref):
        xv = x_ref[...]
        y_ref[...] = jnp.dot(xv, wl_ref[...], preferred_element_type=jnp.float32)
        r_ref[...] = (jnp.dot(xv, wr_ref[...], preferred_element_type=jnp.float32)
                      + b_ref[...][None, :])

    return pl.pallas_call(
        body,
        out_shape=(jax.ShapeDtypeStruct((N, Do), jnp.float32),
                   jax.ShapeDtypeStruct((N, Do), jnp.float32)),
    )(x, Wl, Wr, b)


def _tc_mid1(p, cntp, r, Wl, Wr, b):
    Do = Wl.shape[1]

    def body(p_ref, c_ref, r_ref, wl_ref, wr_ref, b_ref,
             y_ref, rn_ref, inv_ref):
        sacc = p_ref[0:N] + p_ref[NACC:NACC + N]
        cnt = c_ref[0:N, 0:1] + c_ref[NACC:NACC + N, 0:1]
        inv = 1.0 / jnp.maximum(cnt, 1.0)
        inv_ref[...] = inv
        h = jnp.maximum(sacc * inv + r_ref[...], 0.0)
        y_ref[...] = jnp.dot(h, wl_ref[...], preferred_element_type=jnp.float32)
        rn_ref[...] = (jnp.dot(h, wr_ref[...], preferred_element_type=jnp.float32)
                       + b_ref[...][None, :])

    return pl.pallas_call(
        body,
        out_shape=(jax.ShapeDtypeStruct((N, Do), jnp.float32),
                   jax.ShapeDtypeStruct((N, Do), jnp.float32),
                   jax.ShapeDtypeStruct((N, 1), jnp.float32)),
    )(p, cntp, r, Wl, Wr, b)


def _tc_mid2(p, inv, r, Wl, Wr, b):
    Do = Wl.shape[1]

    def body(p_ref, i_ref, r_ref, wl_ref, wr_ref, b_ref, y_ref, rn_ref):
        sacc = p_ref[0:N] + p_ref[NACC:NACC + N]
        h = jnp.maximum(sacc * i_ref[...] + r_ref[...], 0.0)
        y_ref[...] = jnp.dot(h, wl_ref[...], preferred_element_type=jnp.float32)
        rn_ref[...] = (jnp.dot(h, wr_ref[...], preferred_element_type=jnp.float32)
                       + b_ref[...][None, :])

    return pl.pallas_call(
        body,
        out_shape=(jax.ShapeDtypeStruct((N, Do), jnp.float32),
                   jax.ShapeDtypeStruct((N, Do), jnp.float32)),
    )(p, inv, r, Wl, Wr, b)


def _tc_final(p, inv, r, Wreg, breg):
    def body(p_ref, i_ref, r_ref, w_ref, b_ref, o_ref):
        sacc = p_ref[0:N] + p_ref[NACC:NACC + N]
        h = jnp.maximum(sacc * i_ref[...] + r_ref[...], 0.0)
        o_ref[...] = jnp.sum(h * w_ref[...][:, 0][None, :], axis=1) + b_ref[0]

    return pl.pallas_call(
        body,
        out_shape=jax.ShapeDtypeStruct((N,), jnp.float32),
    )(p, inv, r, Wreg, breg)


def kernel(x, edge_index, W1l, b1l, W1r, W2l, b2l, W2r, W3l, b3l, W3r,
           Wreg, breg):
    eidx2 = edge_index.reshape(2, R, CH)

    zeros64 = jnp.zeros((NACC, 64), jnp.float32)
    zeros32 = jnp.zeros((NACC, 32), jnp.float32)
    zeros16 = jnp.zeros((NACC, 16), jnp.float32)
    ones16 = jnp.ones((CH, 16), jnp.float32)

    y1, r1 = _tc_first(x, W1l, W1r, b1l)
    p1, cntp = _seg_sum_call(y1, eidx2, zeros64, zeros16, ones16)
    y2, r2, inv = _tc_mid1(p1, cntp, r1, W2l, W2r, b2l)
    p2 = _seg_sum_call(y2, eidx2, zeros64)
    y3, r3 = _tc_mid2(p2, inv, r2, W3l, W3r, b3l)
    p3 = _seg_sum_call(y3, eidx2, zeros32)
    return _tc_final(p3, inv, r3, Wreg, breg)
